# Initial kernel scaffold; baseline (speedup 1.0000x reference)
import jax
import jax.numpy as jnp
from jax.experimental import pallas as pl


def kernel(curr_h, next_h, curr_inc, edge_index, W_conv, b_conv, W_fus, b_fus, conv_w, topdown_w, ln_g, ln_b):
    raise NotImplementedError("write your pallas kernel here")


# speedup vs baseline: 8.5864x; 8.5864x over previous
import functools

import jax
import jax.numpy as jnp
from jax import lax
from jax.experimental import pallas as pl
from jax.experimental.pallas import tpu as pltpu
from jax.experimental.pallas import tpu_sc as plsc

N = 10000
E = 320000
D = 128
M = 512

NP = 10240
TRASH = NP - N
NTILES = 16
CH = 160
EPT = CH * 128
EPAD = EPT * NTILES
ROWS_PT = NP // NTILES
IB = 32

_SC_MESH = dict(core_axis_name="c", subcore_axis_name="s")


def _deg_body(idx_hbm, deg_hbm, idx_v, ones_v, zero_v, deg_sh):
    c = lax.axis_index("c")
    s = lax.axis_index("s")

    def init_zero(i, carry):
        zero_v[pl.ds(i * 16, 16)] = jnp.zeros((16,), jnp.float32)
        return carry

    lax.fori_loop(0, ROWS_PT // 16, init_zero, 0)

    def init_one(i, carry):
        ones_v[pl.ds(i * 16, 16)] = jnp.ones((16,), jnp.float32)
        return carry

    lax.fori_loop(0, 128 // 16, init_one, 0)

    pltpu.sync_copy(zero_v, deg_sh.at[pl.ds(s * ROWS_PT, ROWS_PT)])
    pltpu.sync_copy(idx_hbm.at[c, s], idx_v)
    plsc.subcore_barrier()

    def chunk(ch, carry):
        pltpu.sync_copy(ones_v, deg_sh.at[idx_v.at[ch]], add=True)
        return carry

    lax.fori_loop(0, CH, chunk, 0)
    plsc.subcore_barrier()
    pltpu.sync_copy(deg_sh.at[pl.ds(s * ROWS_PT, ROWS_PT)],
                    deg_hbm.at[pl.ds(c * NP + s * ROWS_PT, ROWS_PT)])


_deg_kernel = functools.partial(
    pl.kernel,
    out_type=jax.ShapeDtypeStruct((2 * NP,), jnp.float32),
    mesh=plsc.VectorSubcoreMesh(**_SC_MESH),
    scratch_types=[
        pltpu.VMEM((CH, 128), jnp.int32),
        pltpu.VMEM((128,), jnp.float32),
        pltpu.VMEM((ROWS_PT,), jnp.float32),
        pltpu.VMEM_SHARED((NP,), jnp.float32),
    ],
)(_deg_body)


def _agg_body(x_hbm, src_hbm, dst_hbm, out_hbm, src_v, dst_v, buf, accum_sh,
              sem):
    c = lax.axis_index("c")
    s = lax.axis_index("s")
    base = c * NP + s * ROWS_PT

    pltpu.sync_copy(x_hbm.at[pl.ds(base, ROWS_PT)],
                    accum_sh.at[pl.ds(s * ROWS_PT, ROWS_PT)])
    plsc.subcore_barrier()

    def outer(blk, carry):
        pltpu.sync_copy(src_hbm.at[c, s, pl.ds(blk * IB, IB)], src_v)
        pltpu.sync_copy(dst_hbm.at[s, pl.ds(blk * IB, IB)], dst_v)

        def inner(j, carry2):
            pltpu.async_copy(x_hbm.at[src_v.at[j]], buf, sem).wait()
            pltpu.sync_copy(buf, accum_sh.at[dst_v.at[j]], add=True)
            return carry2

        lax.fori_loop(0, IB, inner, 0)
        return carry

    lax.fori_loop(0, CH // IB, outer, 0)
    plsc.subcore_barrier()
    pltpu.sync_copy(accum_sh.at[pl.ds(s * ROWS_PT, ROWS_PT)],
                    out_hbm.at[pl.ds(base, ROWS_PT)])


_agg_kernel = functools.partial(
    pl.kernel,
    out_type=jax.ShapeDtypeStruct((2 * NP, 128), jnp.float32),
    mesh=plsc.VectorSubcoreMesh(**_SC_MESH),
    scratch_types=[
        pltpu.VMEM((IB, 128), jnp.int32),
        pltpu.VMEM((IB, 128), jnp.int32),
        pltpu.VMEM((128, 128), jnp.float32),
        pltpu.VMEM_SHARED((NP, 128), jnp.float32),
        pltpu.SemaphoreType.DMA,
    ],
)(_agg_body)


def _scale_body(h_ref, inc_ref, nh_ref, deg_ref, x0_ref, x1_ref):
    scale = lax.rsqrt(deg_ref[...] + 1.0)
    x0_ref[...] = h_ref[...] * scale
    fused = jnp.dot(inc_ref[...], nh_ref[...],
                    preferred_element_type=jnp.float32,
                    precision=lax.Precision.HIGHEST)
    x1_ref[...] = fused * scale


def _tc_scale(curr_h, curr_inc, next_h, deg_out_hist):
    R = 400
    grid = N // R
    return pl.pallas_call(
        _scale_body,
        grid=(grid,),
        in_specs=[
            pl.BlockSpec((R, D), lambda i: (i, 0)),
            pl.BlockSpec((R, M), lambda i: (i, 0)),
            pl.BlockSpec((M, D), lambda i: (0, 0)),
            pl.BlockSpec((R, 1), lambda i: (i, 0)),
        ],
        out_specs=[
            pl.BlockSpec((R, D), lambda i: (i, 0)),
            pl.BlockSpec((R, D), lambda i: (i, 0)),
        ],
        out_shape=[
            jax.ShapeDtypeStruct((NP, D), jnp.float32),
            jax.ShapeDtypeStruct((NP, D), jnp.float32),
        ],
    )(curr_h, curr_inc, next_h, deg_out_hist)


def _final_body(a0_ref, a1_ref, deg_ref, wc_ref, wf_ref, cw_ref, tw_ref,
                bc_ref, bf_ref, g_ref, b_ref, out_ref):
    scale = lax.rsqrt(deg_ref[...] + 1.0)
    a0 = a0_ref[...] * scale
    a1 = a1_ref[...] * scale
    w0 = wc_ref[...] * cw_ref[...]
    w1 = wf_ref[...] * tw_ref[...]
    pre = (jnp.dot(a0, w0, preferred_element_type=jnp.float32,
                   precision=lax.Precision.HIGHEST)
           + jnp.dot(a1, w1, preferred_element_type=jnp.float32,
                     precision=lax.Precision.HIGHEST)
           + bc_ref[...] * cw_ref[...] + bf_ref[...] * tw_ref[...])
    mu = jnp.mean(pre, axis=1, keepdims=True)
    xc = pre - mu
    var = jnp.mean(xc * xc, axis=1, keepdims=True)
    y = xc * lax.rsqrt(var + 1e-5) * g_ref[...] + b_ref[...]
    out_ref[...] = jnp.maximum(y, 0.0)


def _tc_final(agg, deg_in_hist, W_conv, W_fus, conv_w, topdown_w,
              b_conv, b_fus, ln_g, ln_b):
    R = 80
    grid = N // R
    half_off = NP // R
    full = pl.BlockSpec((D, D), lambda i: (0, 0))
    vec = pl.BlockSpec((1, D), lambda i: (0, 0))
    return pl.pallas_call(
        _final_body,
        grid=(grid,),
        in_specs=[
            pl.BlockSpec((R, D), lambda i: (i, 0)),
            pl.BlockSpec((R, D), lambda i: (i + half_off, 0)),
            pl.BlockSpec((R, 1), lambda i: (i, 0)),
            full, full, vec, vec, vec, vec, vec, vec,
        ],
        out_specs=pl.BlockSpec((R, D), lambda i: (i, 0)),
        out_shape=jax.ShapeDtypeStruct((N, D), jnp.float32),
    )(agg, agg, deg_in_hist, W_conv, W_fus, conv_w.reshape(1, D),
      topdown_w.reshape(1, D), b_conv.reshape(1, D), b_fus.reshape(1, D),
      ln_g.reshape(1, D), ln_b.reshape(1, D))


def kernel(curr_h, next_h, curr_inc, edge_index, W_conv, b_conv, W_fus, b_fus,
           conv_w, topdown_w, ln_g, ln_b):
    ei = edge_index.astype(jnp.int32)
    src, dst = ei[0], ei[1]
    pad = EPAD - E
    trash = N + (jnp.arange(pad, dtype=jnp.int32) % TRASH)
    src_p = jnp.concatenate([src, trash]).reshape(NTILES, CH, 128)
    dst_p = jnp.concatenate([dst, trash]).reshape(NTILES, CH, 128)

    idx4 = jnp.stack([src_p, dst_p])
    deg2 = _deg_kernel(idx4)

    x0, x1 = _tc_scale(curr_h, curr_inc, next_h, deg2[:N].reshape(N, 1))
    x = jnp.concatenate([x0, x1], axis=0)

    src4 = jnp.stack([src_p, src_p + NP])
    agg = _agg_kernel(x, src4, dst_p)

    return _tc_final(agg, deg2[NP:NP + N].reshape(N, 1),
                     W_conv, W_fus, conv_w, topdown_w, b_conv, b_fus,
                     ln_g, ln_b)


# speedup vs baseline: 8.9241x; 1.0393x over previous
import functools

import jax
import jax.numpy as jnp
from jax import lax
from jax.experimental import pallas as pl
from jax.experimental.pallas import tpu as pltpu
from jax.experimental.pallas import tpu_sc as plsc

N = 10000
E = 320000
D = 128
M = 512

NP = 10240
TRASH = NP - N
NTILES = 16
CH = 160
EPT = CH * 128
EPAD = EPT * NTILES
ROWS_PT = NP // NTILES
IB = 16

_SC_MESH = dict(core_axis_name="c", subcore_axis_name="s")


def _deg_body(idx_hbm, deg_hbm, idx_v, ones_v, zero_v, deg_sh):
    c = lax.axis_index("c")
    s = lax.axis_index("s")

    def init_zero(i, carry):
        zero_v[pl.ds(i * 16, 16)] = jnp.zeros((16,), jnp.float32)
        return carry

    lax.fori_loop(0, ROWS_PT // 16, init_zero, 0)

    def init_one(i, carry):
        ones_v[pl.ds(i * 16, 16)] = jnp.ones((16,), jnp.float32)
        return carry

    lax.fori_loop(0, 128 // 16, init_one, 0)

    pltpu.sync_copy(zero_v, deg_sh.at[pl.ds(s * ROWS_PT, ROWS_PT)])
    pltpu.sync_copy(idx_hbm.at[c, s], idx_v)
    plsc.subcore_barrier()

    def chunk(ch, carry):
        pltpu.sync_copy(ones_v, deg_sh.at[idx_v.at[ch]], add=True)
        return carry

    lax.fori_loop(0, CH, chunk, 0)
    plsc.subcore_barrier()
    pltpu.sync_copy(deg_sh.at[pl.ds(s * ROWS_PT, ROWS_PT)],
                    deg_hbm.at[pl.ds(c * NP + s * ROWS_PT, ROWS_PT)])


_deg_kernel = functools.partial(
    pl.kernel,
    out_type=jax.ShapeDtypeStruct((2 * NP,), jnp.float32),
    mesh=plsc.VectorSubcoreMesh(**_SC_MESH),
    scratch_types=[
        pltpu.VMEM((CH, 128), jnp.int32),
        pltpu.VMEM((128,), jnp.float32),
        pltpu.VMEM((ROWS_PT,), jnp.float32),
        pltpu.VMEM_SHARED((NP,), jnp.float32),
    ],
)(_deg_body)


def _agg_body(x_hbm, src_hbm, dst_hbm, out_hbm, src_v, dst_v, buf0, buf1,
              accum_sh, sem0, sem1):
    c = lax.axis_index("c")
    s = lax.axis_index("s")
    base = c * NP + s * ROWS_PT

    pltpu.sync_copy(x_hbm.at[pl.ds(base, ROWS_PT)],
                    accum_sh.at[pl.ds(s * ROWS_PT, ROWS_PT)])
    plsc.subcore_barrier()

    bufs = (buf0, buf1)
    sems = (sem0, sem1)

    def block(blk, carry):
        pltpu.sync_copy(src_hbm.at[c, s, pl.ds(blk * IB, IB)], src_v)
        pltpu.sync_copy(dst_hbm.at[s, pl.ds(blk * IB, IB)], dst_v)
        descs = {0: pltpu.async_copy(x_hbm.at[src_v.at[0]], buf0, sem0)}
        for j in range(IB):
            if j + 1 < IB:
                descs[j + 1] = pltpu.async_copy(
                    x_hbm.at[src_v.at[j + 1]], bufs[(j + 1) % 2],
                    sems[(j + 1) % 2])
            descs[j].wait()
            pltpu.sync_copy(bufs[j % 2], accum_sh.at[dst_v.at[j]], add=True)
        return carry

    lax.fori_loop(0, CH // IB, block, 0)
    plsc.subcore_barrier()
    pltpu.sync_copy(accum_sh.at[pl.ds(s * ROWS_PT, ROWS_PT)],
                    out_hbm.at[pl.ds(base, ROWS_PT)])


_agg_kernel = functools.partial(
    pl.kernel,
    out_type=jax.ShapeDtypeStruct((2 * NP, 128), jnp.float32),
    mesh=plsc.VectorSubcoreMesh(**_SC_MESH),
    scratch_types=[
        pltpu.VMEM((IB, 128), jnp.int32),
        pltpu.VMEM((IB, 128), jnp.int32),
        pltpu.VMEM((128, 128), jnp.float32),
        pltpu.VMEM((128, 128), jnp.float32),
        pltpu.VMEM_SHARED((NP, 128), jnp.float32),
        pltpu.SemaphoreType.DMA,
        pltpu.SemaphoreType.DMA,
    ],
)(_agg_body)


def _scale_body(h_ref, inc_ref, nh_ref, deg_ref, x_ref):
    g = pl.program_id(1)
    scale = lax.rsqrt(deg_ref[...] + 1.0)

    @pl.when(g == 0)
    def _():
        x_ref[...] = h_ref[...] * scale

    @pl.when(g == 1)
    def _():
        fused = jnp.dot(inc_ref[...], nh_ref[...],
                        preferred_element_type=jnp.float32,
                        precision=lax.Precision.HIGHEST)
        x_ref[...] = fused * scale


def _tc_scale(curr_h, curr_inc, next_h, deg_out_hist):
    R = 80
    half_off = NP // R
    return pl.pallas_call(
        _scale_body,
        grid=(N // R, 2),
        in_specs=[
            pl.BlockSpec((R, D), lambda i, g: (i, 0)),
            pl.BlockSpec((R, M), lambda i, g: (i, 0)),
            pl.BlockSpec((M, D), lambda i, g: (0, 0)),
            pl.BlockSpec((R, 1), lambda i, g: (i, 0)),
        ],
        out_specs=pl.BlockSpec((R, D), lambda i, g: (g * half_off + i, 0)),
        out_shape=jax.ShapeDtypeStruct((2 * NP, D), jnp.float32),
    )(curr_h, curr_inc, next_h, deg_out_hist)


def _final_body(a0_ref, a1_ref, deg_ref, wc_ref, wf_ref, cw_ref, tw_ref,
                bc_ref, bf_ref, g_ref, b_ref, out_ref):
    scale = lax.rsqrt(deg_ref[...] + 1.0)
    a0 = a0_ref[...] * scale
    a1 = a1_ref[...] * scale
    w0 = wc_ref[...] * cw_ref[...]
    w1 = wf_ref[...] * tw_ref[...]
    pre = (jnp.dot(a0, w0, preferred_element_type=jnp.float32,
                   precision=lax.Precision.HIGHEST)
           + jnp.dot(a1, w1, preferred_element_type=jnp.float32,
                     precision=lax.Precision.HIGHEST)
           + bc_ref[...] * cw_ref[...] + bf_ref[...] * tw_ref[...])
    mu = jnp.mean(pre, axis=1, keepdims=True)
    xc = pre - mu
    var = jnp.mean(xc * xc, axis=1, keepdims=True)
    y = xc * lax.rsqrt(var + 1e-5) * g_ref[...] + b_ref[...]
    out_ref[...] = jnp.maximum(y, 0.0)


def _tc_final(agg, deg_in_hist, W_conv, W_fus, conv_w, topdown_w,
              b_conv, b_fus, ln_g, ln_b):
    R = 80
    grid = N // R
    half_off = NP // R
    full = pl.BlockSpec((D, D), lambda i: (0, 0))
    vec = pl.BlockSpec((1, D), lambda i: (0, 0))
    return pl.pallas_call(
        _final_body,
        grid=(grid,),
        in_specs=[
            pl.BlockSpec((R, D), lambda i: (i, 0)),
            pl.BlockSpec((R, D), lambda i: (i + half_off, 0)),
            pl.BlockSpec((R, 1), lambda i: (i, 0)),
            full, full, vec, vec, vec, vec, vec, vec,
        ],
        out_specs=pl.BlockSpec((R, D), lambda i: (i, 0)),
        out_shape=jax.ShapeDtypeStruct((N, D), jnp.float32),
    )(agg, agg, deg_in_hist, W_conv, W_fus, conv_w.reshape(1, D),
      topdown_w.reshape(1, D), b_conv.reshape(1, D), b_fus.reshape(1, D),
      ln_g.reshape(1, D), ln_b.reshape(1, D))


def kernel(curr_h, next_h, curr_inc, edge_index, W_conv, b_conv, W_fus, b_fus,
           conv_w, topdown_w, ln_g, ln_b):
    ei = edge_index.astype(jnp.int32)
    src, dst = ei[0], ei[1]
    pad = EPAD - E
    trash = N + (jnp.arange(pad, dtype=jnp.int32) % TRASH)
    src_p = jnp.concatenate([src, trash]).reshape(NTILES, CH, 128)
    dst_p = jnp.concatenate([dst, trash]).reshape(NTILES, CH, 128)

    idx4 = jnp.stack([src_p, dst_p])
    deg2 = _deg_kernel(idx4)

    x = _tc_scale(curr_h, curr_inc, next_h, deg2[:N].reshape(N, 1))

    src4 = jnp.stack([src_p, src_p + NP])
    agg = _agg_kernel(x, src4, dst_p)

    return _tc_final(agg, deg2[NP:NP + N].reshape(N, 1),
                     W_conv, W_fus, conv_w, topdown_w, b_conv, b_fus,
                     ln_g, ln_b)


# speedup vs baseline: 14.0028x; 1.5691x over previous
import functools

import jax
import jax.numpy as jnp
from jax import lax
from jax.experimental import pallas as pl
from jax.experimental.pallas import tpu as pltpu
from jax.experimental.pallas import tpu_sc as plsc

N = 10000
E = 320000
D = 128
M = 512

NP = 10240
TRASH = NP - N
NTILES = 16
CH = 160
EPT = CH * 128
EPAD = EPT * NTILES
ROWS_PT = NP // NTILES
IB = 16

_SC_MESH = dict(core_axis_name="c", subcore_axis_name="s")


def _deg_body(idx_hbm, deg_hbm, idx_v, ones_v, zero_v, deg_sh):
    c = lax.axis_index("c")
    s = lax.axis_index("s")

    def init_zero(i, carry):
        zero_v[pl.ds(i * 16, 16)] = jnp.zeros((16,), jnp.float32)
        return carry

    lax.fori_loop(0, ROWS_PT // 16, init_zero, 0)

    def init_one(i, carry):
        ones_v[pl.ds(i * 16, 16)] = jnp.ones((16,), jnp.float32)
        return carry

    lax.fori_loop(0, 128 // 16, init_one, 0)

    pltpu.sync_copy(zero_v, deg_sh.at[pl.ds(s * ROWS_PT, ROWS_PT)])
    pltpu.sync_copy(idx_hbm.at[c, s], idx_v)
    plsc.subcore_barrier()

    def chunk(ch, carry):
        pltpu.sync_copy(ones_v, deg_sh.at[idx_v.at[ch]], add=True)
        return carry

    lax.fori_loop(0, CH, chunk, 0)
    plsc.subcore_barrier()
    pltpu.sync_copy(deg_sh.at[pl.ds(s * ROWS_PT, ROWS_PT)],
                    deg_hbm.at[pl.ds(c * NP + s * ROWS_PT, ROWS_PT)])


_deg_kernel = functools.partial(
    pl.kernel,
    out_type=jax.ShapeDtypeStruct((2 * NP,), jnp.float32),
    mesh=plsc.VectorSubcoreMesh(**_SC_MESH),
    scratch_types=[
        pltpu.VMEM((CH, 128), jnp.int32),
        pltpu.VMEM((128,), jnp.float32),
        pltpu.VMEM((ROWS_PT,), jnp.float32),
        pltpu.VMEM_SHARED((NP,), jnp.float32),
    ],
)(_deg_body)


def _agg_body(x_hbm, idx_hbm, out_hbm, src_v, dst_v, buf0, buf1,
              accum_sh, sem0, sem1):
    c = lax.axis_index("c")
    s = lax.axis_index("s")
    base = c * NP + s * ROWS_PT

    pltpu.sync_copy(x_hbm.at[pl.ds(base, ROWS_PT)],
                    accum_sh.at[pl.ds(s * ROWS_PT, ROWS_PT)])
    plsc.subcore_barrier()

    bufs = (buf0, buf1)
    sems = (sem0, sem1)

    def block(blk, carry):
        pltpu.sync_copy(idx_hbm.at[2 * c, s, pl.ds(blk * IB, IB)], src_v)
        pltpu.sync_copy(idx_hbm.at[1, s, pl.ds(blk * IB, IB)], dst_v)
        descs = {0: pltpu.async_copy(x_hbm.at[src_v.at[0]], buf0, sem0)}
        for j in range(IB):
            if j + 1 < IB:
                descs[j + 1] = pltpu.async_copy(
                    x_hbm.at[src_v.at[j + 1]], bufs[(j + 1) % 2],
                    sems[(j + 1) % 2])
            descs[j].wait()
            pltpu.sync_copy(bufs[j % 2], accum_sh.at[dst_v.at[j]], add=True)
        return carry

    lax.fori_loop(0, CH // IB, block, 0)
    plsc.subcore_barrier()
    pltpu.sync_copy(accum_sh.at[pl.ds(s * ROWS_PT, ROWS_PT)],
                    out_hbm.at[pl.ds(base, ROWS_PT)])


_agg_kernel = functools.partial(
    pl.kernel,
    out_type=jax.ShapeDtypeStruct((2 * NP, 128), jnp.float32),
    mesh=plsc.VectorSubcoreMesh(**_SC_MESH),
    scratch_types=[
        pltpu.VMEM((IB, 128), jnp.int32),
        pltpu.VMEM((IB, 128), jnp.int32),
        pltpu.VMEM((128, 128), jnp.float32),
        pltpu.VMEM((128, 128), jnp.float32),
        pltpu.VMEM_SHARED((NP, 128), jnp.float32),
        pltpu.SemaphoreType.DMA,
        pltpu.SemaphoreType.DMA,
    ],
)(_agg_body)


def _scale_body(h_ref, inc_ref, nh_ref, deg_ref, x_ref):
    scale = lax.rsqrt(deg_ref[...] + 1.0)
    x_ref[0] = h_ref[...] * scale
    fused = jnp.dot(inc_ref[...], nh_ref[...],
                    preferred_element_type=jnp.float32,
                    precision=lax.Precision.HIGHEST)
    x_ref[1] = fused * scale


def _tc_scale(curr_h, curr_inc, next_h, deg_out_hist):
    R = 1000
    return pl.pallas_call(
        _scale_body,
        grid=(N // R,),
        in_specs=[
            pl.BlockSpec((R, D), lambda i: (i, 0)),
            pl.BlockSpec((R, M), lambda i: (i, 0)),
            pl.BlockSpec((M, D), lambda i: (0, 0)),
            pl.BlockSpec((R, 1), lambda i: (i, 0)),
        ],
        out_specs=pl.BlockSpec((2, R, D), lambda i: (0, i, 0)),
        out_shape=jax.ShapeDtypeStruct((2, NP, D), jnp.float32),
    )(curr_h, curr_inc, next_h, deg_out_hist)


def _final_body(agg_ref, deg_ref, wc_ref, wf_ref, cw_ref, tw_ref,
                bc_ref, bf_ref, g_ref, b_ref, out_ref):
    scale = lax.rsqrt(deg_ref[...] + 1.0)
    a0 = agg_ref[0] * scale
    a1 = agg_ref[1] * scale
    w0 = wc_ref[...] * cw_ref[...]
    w1 = wf_ref[...] * tw_ref[...]
    pre = (jnp.dot(a0, w0, preferred_element_type=jnp.float32,
                   precision=lax.Precision.HIGHEST)
           + jnp.dot(a1, w1, preferred_element_type=jnp.float32,
                     precision=lax.Precision.HIGHEST)
           + bc_ref[...] * cw_ref[...] + bf_ref[...] * tw_ref[...])
    mu = jnp.mean(pre, axis=1, keepdims=True)
    xc = pre - mu
    var = jnp.mean(xc * xc, axis=1, keepdims=True)
    y = xc * lax.rsqrt(var + 1e-5) * g_ref[...] + b_ref[...]
    out_ref[...] = jnp.maximum(y, 0.0)


def _tc_final(agg, deg_in_hist, W_conv, W_fus, conv_w, topdown_w,
              b_conv, b_fus, ln_g, ln_b):
    R = 640
    grid = NP // R
    full = pl.BlockSpec((D, D), lambda i: (0, 0))
    vec = pl.BlockSpec((1, D), lambda i: (0, 0))
    out = pl.pallas_call(
        _final_body,
        grid=(grid,),
        in_specs=[
            pl.BlockSpec((2, R, D), lambda i: (0, i, 0)),
            pl.BlockSpec((R, 1), lambda i: (i, 0)),
            full, full, vec, vec, vec, vec, vec, vec,
        ],
        out_specs=pl.BlockSpec((R, D), lambda i: (i, 0)),
        out_shape=jax.ShapeDtypeStruct((NP, D), jnp.float32),
    )(agg, deg_in_hist, W_conv, W_fus, conv_w.reshape(1, D),
      topdown_w.reshape(1, D), b_conv.reshape(1, D), b_fus.reshape(1, D),
      ln_g.reshape(1, D), ln_b.reshape(1, D))
    return out[:N]


def kernel(curr_h, next_h, curr_inc, edge_index, W_conv, b_conv, W_fus, b_fus,
           conv_w, topdown_w, ln_g, ln_b):
    ei = edge_index.astype(jnp.int32)
    src, dst = ei[0], ei[1]
    pad = EPAD - E
    trash = N + (jnp.arange(pad, dtype=jnp.int32) % TRASH)
    src_p = jnp.concatenate([src, trash]).reshape(NTILES, CH, 128)
    dst_p = jnp.concatenate([dst, trash]).reshape(NTILES, CH, 128)

    combo = jnp.stack([src_p, dst_p, src_p + NP])
    deg2 = _deg_kernel(combo)

    x = _tc_scale(curr_h, curr_inc, next_h, deg2[:N].reshape(N, 1))

    agg = _agg_kernel(x.reshape(2 * NP, D), combo)

    return _tc_final(agg.reshape(2, NP, D), deg2[NP:].reshape(NP, 1),
                     W_conv, W_fus, conv_w, topdown_w, b_conv, b_fus,
                     ln_g, ln_b)


# speedup vs baseline: 14.3079x; 1.0218x over previous
import functools

import jax
import jax.numpy as jnp
from jax import lax
from jax.experimental import pallas as pl
from jax.experimental.pallas import tpu as pltpu
from jax.experimental.pallas import tpu_sc as plsc

N = 10000
E = 320000
D = 128
M = 512

NP = 10240
TRASH = NP - N
NTILES = 16
CH = 160
EPT = CH * 128
EPAD = EPT * NTILES
ROWS_PT = NP // NTILES
IB = 16

_SC_MESH = dict(core_axis_name="c", subcore_axis_name="s")


def _deg_body(idx_hbm, deg_hbm, idx_v, ones_v, zero_v, deg_sh, sem):
    c = lax.axis_index("c")
    s = lax.axis_index("s")

    def init_zero(i, carry):
        zero_v[pl.ds(i * 16, 16)] = jnp.zeros((16,), jnp.float32)
        return carry

    lax.fori_loop(0, ROWS_PT // 16, init_zero, 0)

    def init_one(i, carry):
        ones_v[pl.ds(i * 16, 16)] = jnp.ones((16,), jnp.float32)
        return carry

    lax.fori_loop(0, 128 // 16, init_one, 0)

    pltpu.sync_copy(zero_v, deg_sh.at[pl.ds(s * ROWS_PT, ROWS_PT)])
    pltpu.sync_copy(idx_hbm.at[c, s], idx_v)
    plsc.subcore_barrier()

    DB = 16

    def batch(b, carry):
        descs = [
            pltpu.async_copy(ones_v, deg_sh.at[idx_v.at[b * DB + k]], sem,
                             add=True)
            for k in range(DB)
        ]
        for d in descs:
            d.wait()
        return carry

    lax.fori_loop(0, CH // DB, batch, 0)
    plsc.subcore_barrier()
    pltpu.sync_copy(deg_sh.at[pl.ds(s * ROWS_PT, ROWS_PT)],
                    deg_hbm.at[pl.ds(c * NP + s * ROWS_PT, ROWS_PT)])


_deg_kernel = functools.partial(
    pl.kernel,
    out_type=jax.ShapeDtypeStruct((2 * NP,), jnp.float32),
    mesh=plsc.VectorSubcoreMesh(**_SC_MESH),
    scratch_types=[
        pltpu.VMEM((CH, 128), jnp.int32),
        pltpu.VMEM((128,), jnp.float32),
        pltpu.VMEM((ROWS_PT,), jnp.float32),
        pltpu.VMEM_SHARED((NP,), jnp.float32),
        pltpu.SemaphoreType.DMA,
    ],
)(_deg_body)


def _agg_body(x_hbm, idx_hbm, out_hbm, src_v, dst_v, buf0, buf1,
              accum_sh, sem0, sem1):
    c = lax.axis_index("c")
    s = lax.axis_index("s")
    xc = x_hbm.at[c]

    pltpu.sync_copy(xc.at[pl.ds(s * ROWS_PT, ROWS_PT)],
                    accum_sh.at[pl.ds(s * ROWS_PT, ROWS_PT)])
    plsc.subcore_barrier()

    bufs = (buf0, buf1)
    sems = (sem0, sem1)

    def block(blk, carry):
        pltpu.sync_copy(idx_hbm.at[0, s, pl.ds(blk * IB, IB)], src_v)
        pltpu.sync_copy(idx_hbm.at[1, s, pl.ds(blk * IB, IB)], dst_v)
        descs = {0: pltpu.async_copy(xc.at[src_v.at[0]], buf0, sem0)}
        for j in range(IB):
            if j + 1 < IB:
                descs[j + 1] = pltpu.async_copy(
                    xc.at[src_v.at[j + 1]], bufs[(j + 1) % 2],
                    sems[(j + 1) % 2])
            descs[j].wait()
            pltpu.sync_copy(bufs[j % 2], accum_sh.at[dst_v.at[j]], add=True)
        return carry

    lax.fori_loop(0, CH // IB, block, 0)
    plsc.subcore_barrier()
    pltpu.sync_copy(accum_sh.at[pl.ds(s * ROWS_PT, ROWS_PT)],
                    out_hbm.at[c, pl.ds(s * ROWS_PT, ROWS_PT)])


_agg_kernel = functools.partial(
    pl.kernel,
    out_type=jax.ShapeDtypeStruct((2, NP, 128), jnp.float32),
    mesh=plsc.VectorSubcoreMesh(**_SC_MESH),
    scratch_types=[
        pltpu.VMEM((IB, 128), jnp.int32),
        pltpu.VMEM((IB, 128), jnp.int32),
        pltpu.VMEM((128, 128), jnp.float32),
        pltpu.VMEM((128, 128), jnp.float32),
        pltpu.VMEM_SHARED((NP, 128), jnp.float32),
        pltpu.SemaphoreType.DMA,
        pltpu.SemaphoreType.DMA,
    ],
)(_agg_body)


def _scale_body(h_ref, inc_ref, nh_ref, deg_ref, x_ref):
    scale = lax.rsqrt(deg_ref[...] + 1.0)
    x_ref[0] = h_ref[...] * scale
    fused = jnp.dot(inc_ref[...], nh_ref[...],
                    preferred_element_type=jnp.float32,
                    precision=lax.Precision.HIGHEST)
    x_ref[1] = fused * scale


def _tc_scale(curr_h, curr_inc, next_h, deg_out_hist):
    R = 1000
    return pl.pallas_call(
        _scale_body,
        grid=(N // R,),
        in_specs=[
            pl.BlockSpec((R, D), lambda i: (i, 0)),
            pl.BlockSpec((R, M), lambda i: (i, 0)),
            pl.BlockSpec((M, D), lambda i: (0, 0)),
            pl.BlockSpec((R, 1), lambda i: (i, 0)),
        ],
        out_specs=pl.BlockSpec((2, R, D), lambda i: (0, i, 0)),
        out_shape=jax.ShapeDtypeStruct((2, NP, D), jnp.float32),
    )(curr_h, curr_inc, next_h, deg_out_hist)


def _final_body(agg_ref, deg_ref, wc_ref, wf_ref, cw_ref, tw_ref,
                bc_ref, bf_ref, g_ref, b_ref, out_ref):
    scale = lax.rsqrt(deg_ref[...] + 1.0)
    a0 = agg_ref[0] * scale
    a1 = agg_ref[1] * scale
    w0 = wc_ref[...] * cw_ref[...]
    w1 = wf_ref[...] * tw_ref[...]
    pre = (jnp.dot(a0, w0, preferred_element_type=jnp.float32,
                   precision=lax.Precision.HIGHEST)
           + jnp.dot(a1, w1, preferred_element_type=jnp.float32,
                     precision=lax.Precision.HIGHEST)
           + bc_ref[...] * cw_ref[...] + bf_ref[...] * tw_ref[...])
    mu = jnp.mean(pre, axis=1, keepdims=True)
    xc = pre - mu
    var = jnp.mean(xc * xc, axis=1, keepdims=True)
    y = xc * lax.rsqrt(var + 1e-5) * g_ref[...] + b_ref[...]
    out_ref[...] = jnp.maximum(y, 0.0)


def _tc_final(agg, deg_in_hist, W_conv, W_fus, conv_w, topdown_w,
              b_conv, b_fus, ln_g, ln_b):
    R = 400
    grid = N // R
    full = pl.BlockSpec((D, D), lambda i: (0, 0))
    vec = pl.BlockSpec((1, D), lambda i: (0, 0))
    return pl.pallas_call(
        _final_body,
        grid=(grid,),
        in_specs=[
            pl.BlockSpec((2, R, D), lambda i: (0, i, 0)),
            pl.BlockSpec((R, 1), lambda i: (i, 0)),
            full, full, vec, vec, vec, vec, vec, vec,
        ],
        out_specs=pl.BlockSpec((R, D), lambda i: (i, 0)),
        out_shape=jax.ShapeDtypeStruct((N, D), jnp.float32),
    )(agg, deg_in_hist, W_conv, W_fus, conv_w.reshape(1, D),
      topdown_w.reshape(1, D), b_conv.reshape(1, D), b_fus.reshape(1, D),
      ln_g.reshape(1, D), ln_b.reshape(1, D))


def kernel(curr_h, next_h, curr_inc, edge_index, W_conv, b_conv, W_fus, b_fus,
           conv_w, topdown_w, ln_g, ln_b):
    ei = edge_index.astype(jnp.int32)
    src, dst = ei[0], ei[1]
    pad = EPAD - E
    trash = N + (jnp.arange(pad, dtype=jnp.int32) % TRASH)
    src_p = jnp.concatenate([src, trash]).reshape(NTILES, CH, 128)
    dst_p = jnp.concatenate([dst, trash]).reshape(NTILES, CH, 128)

    combo = jnp.stack([src_p, dst_p])
    deg2 = _deg_kernel(combo)

    x = _tc_scale(curr_h, curr_inc, next_h, deg2[:N].reshape(N, 1))

    agg = _agg_kernel(x, combo)

    return _tc_final(agg, deg2[NP:].reshape(NP, 1),
                     W_conv, W_fus, conv_w, topdown_w, b_conv, b_fus,
                     ln_g, ln_b)
